# baseline (speedup 1.0000x reference)
import functools

import jax
import jax.numpy as jnp
from jax.experimental import pallas as pl
from jax.experimental.pallas import tpu as pltpu

_T = 2048
_D = 1024
_E = 8
_K = 2
_F = 4096
_EPS = 1e-5
_FB = 512

_NEG = jnp.finfo(jnp.float32).min


def _router_body(x_ref, mask_ref, lnw_ref, wr_ref, p_ref):
    x = x_ref[...]
    var = jnp.mean(x * x, axis=-1, keepdims=True)
    xn = x * jax.lax.rsqrt(var + _EPS) * lnw_ref[...]
    logits = jnp.dot(xn, wr_ref[...], preferred_element_type=jnp.float32)
    mask = mask_ref[...] != 0
    logits = jnp.where(mask, logits, _NEG)
    ii = jax.lax.broadcasted_iota(jnp.int32, logits.shape, 1)
    m1 = jnp.max(logits, axis=-1, keepdims=True)
    i1 = jnp.min(jnp.where(logits == m1, ii, _E), axis=-1, keepdims=True)
    is1 = ii == i1
    l2 = jnp.where(is1, _NEG, logits)
    m2 = jnp.max(l2, axis=-1, keepdims=True)
    i2 = jnp.min(jnp.where(l2 == m2, ii, _E), axis=-1, keepdims=True)
    is2 = ii == i2
    hard = is1 | is2
    z = jnp.exp(logits - m1)
    probs = z / jnp.sum(z, axis=-1, keepdims=True)
    probs = jnp.where(hard & mask, probs, 0.0)
    p_ref[...] = probs


def _router(x, mask, ln_w, w_router):
    return pl.pallas_call(
        _router_body,
        out_shape=jax.ShapeDtypeStruct((_T, _E), jnp.float32),
    )(x, mask.astype(jnp.int32).reshape(_T, 1), ln_w.reshape(1, _D), w_router)


def _moe_body(x_bf_ref, x_res_ref, w1_ref, w2_ref, p_ref, out_ref):
    e = pl.program_id(0)
    fb = pl.program_id(1)

    @pl.when((e == 0) & (fb == 0))
    def _():
        out_ref[...] = x_res_ref[...]

    h = jnp.dot(x_bf_ref[...], w1_ref[0], preferred_element_type=jnp.float32)
    h = jax.nn.gelu(h)
    hb = (h * p_ref[0]).astype(jnp.bfloat16)
    out_ref[...] += jnp.dot(hb, w2_ref[0], preferred_element_type=jnp.float32)


def kernel(x, mask, ln_w, w_router, w1, w2):
    probs = _router(x, mask, ln_w, w_router)
    p_col = probs.T.reshape(_E, _T, 1)
    x_bf = x.astype(jnp.bfloat16)
    w1_bf = w1.astype(jnp.bfloat16)
    w2_bf = w2.astype(jnp.bfloat16)

    grid = (_E, _F // _FB)
    out = pl.pallas_call(
        _moe_body,
        grid=grid,
        in_specs=[
            pl.BlockSpec((_T, _D), lambda e, f: (0, 0)),
            pl.BlockSpec((_T, _D), lambda e, f: (0, 0)),
            pl.BlockSpec((1, _D, _FB), lambda e, f: (e, 0, f)),
            pl.BlockSpec((1, _FB, _D), lambda e, f: (e, f, 0)),
            pl.BlockSpec((1, _T, 1), lambda e, f: (e, 0, 0)),
        ],
        out_specs=pl.BlockSpec((_T, _D), lambda e, f: (0, 0)),
        out_shape=jax.ShapeDtypeStruct((_T, _D), jnp.float32),
    )(x_bf, x, w1_bf, w2_bf, p_col)
    return out
